# baseline (speedup 1.0000x reference)
import functools

import jax
import jax.numpy as jnp
from jax import lax
from jax.experimental import pallas as pl
from jax.experimental.pallas import tpu as pltpu
from jax.experimental.pallas import tpu_sc as plsc

N_CTRL = 32


@functools.cache
def _make_kernel(B, D):
    num_cores = 1
    num_subcores = 4
    n_workers = num_cores * num_subcores
    batches_per_w = B // n_workers
    rows_total = B * N_CTRL
    mesh = plsc.VectorSubcoreMesh(core_axis_name="c", subcore_axis_name="s",
                                  num_cores=num_cores,
                                  num_subcores=num_subcores)

    @functools.partial(
        pl.kernel,
        mesh=mesh,
        out_type=jax.ShapeDtypeStruct((rows_total, D), jnp.float32),
        scratch_types=[
            pltpu.VMEM((N_CTRL, D), jnp.float32),
            pltpu.SemaphoreType.DMA,
        ],
    )
    def tile_copy(table_hbm, out_hbm, buf, sem):
        wid = lax.axis_index("s") * num_cores + lax.axis_index("c")
        pltpu.async_copy(table_hbm.at[pl.ds(0, N_CTRL), :], buf, sem).wait()
        base = wid * batches_per_w * N_CTRL
        copies = []
        for j in range(batches_per_w):
            copies.append(pltpu.async_copy(
                buf, out_hbm.at[pl.ds(base + j * N_CTRL, N_CTRL), :], sem))
        for c in copies:
            c.wait()

    return tile_copy


def kernel(x, embed_table):
    B = x.shape[0]
    D = embed_table.shape[1]
    out_flat = _make_kernel(B, D)(embed_table)
    return out_flat.reshape(B, N_CTRL, D)


# speedup vs baseline: 1.0350x; 1.0350x over previous
import functools

import jax
import jax.numpy as jnp
from jax import lax
from jax.experimental import pallas as pl
from jax.experimental.pallas import tpu as pltpu
from jax.experimental.pallas import tpu_sc as plsc

N_CTRL = 32
NUM_CORES = 1
NUM_SUBCORES = 16


@functools.cache
def _make_kernel(B, D):
    n_workers = NUM_CORES * NUM_SUBCORES
    rows_total = B * N_CTRL
    rows_per_w = rows_total // n_workers
    half = rows_per_w // 2
    mesh = plsc.VectorSubcoreMesh(core_axis_name="c", subcore_axis_name="s",
                                  num_cores=NUM_CORES)

    @functools.partial(
        pl.kernel,
        mesh=mesh,
        out_type=jax.ShapeDtypeStruct((rows_total, D), jnp.float32),
        scratch_types=[
            pltpu.VMEM((rows_per_w, D), jnp.float32),
            pltpu.SemaphoreType.DMA,
            pltpu.SemaphoreType.DMA,
        ],
    )
    def tile_copy(table_hbm, out_hbm, buf, sem_a, sem_b):
        wid = lax.axis_index("s") * NUM_CORES + lax.axis_index("c")
        out_base = wid * rows_per_w
        tab_base = out_base % N_CTRL
        g0 = pltpu.async_copy(table_hbm.at[pl.ds(tab_base, half), :],
                              buf.at[pl.ds(0, half), :], sem_a)
        g1 = pltpu.async_copy(table_hbm.at[pl.ds(tab_base + half, half), :],
                              buf.at[pl.ds(half, half), :], sem_b)
        g0.wait()
        s0 = pltpu.async_copy(buf.at[pl.ds(0, half), :],
                              out_hbm.at[pl.ds(out_base, half), :], sem_a)
        g1.wait()
        s1 = pltpu.async_copy(buf.at[pl.ds(half, half), :],
                              out_hbm.at[pl.ds(out_base + half, half), :],
                              sem_b)
        s0.wait()
        s1.wait()

    return tile_copy


def kernel(x, embed_table):
    B = x.shape[0]
    D = embed_table.shape[1]
    out_flat = _make_kernel(B, D)(embed_table)
    return out_flat.reshape(B, N_CTRL, D)
